# baseline (speedup 1.0000x reference)
import functools

import jax
import jax.numpy as jnp
from jax import lax
from jax.experimental import pallas as pl
from jax.experimental.pallas import tpu as pltpu
from jax.experimental.pallas import tpu_sc as plsc

VOCAB = 1000000
D = 64
BATCH = 16384
HIST = 50
B = BATCH * HIST

_info = plsc.get_sparse_core_info()
NC, NS = _info.num_cores, _info.num_subcores
NW = NC * NS
ROWS_PER_W = BATCH // NW
B_PER_W = B // NW
BB_PER_W = ROWS_PER_W // 128
N_ROUNDS = HIST * BB_PER_W


@functools.partial(
    pl.kernel,
    mesh=plsc.VectorSubcoreMesh(core_axis_name="c", subcore_axis_name="s"),
    out_type=jax.ShapeDtypeStruct((HIST, 8, BATCH // 128, 8, 128), jnp.float32),
    scratch_types=[
        pltpu.VMEM((B_PER_W,), jnp.int32),
        pltpu.VMEM((HIST, BB_PER_W, 128), jnp.int32),
        [pltpu.VMEM((128, D), jnp.float32) for _ in range(2)],
        [pltpu.VMEM((D, 128), jnp.float32) for _ in range(2)],
        [pltpu.SemaphoreType.DMA for _ in range(2)],
        [pltpu.SemaphoreType.DMA for _ in range(2)],
    ],
    compiler_params=pltpu.CompilerParams(use_tc_tiling_on_sc=False, needs_layout_passes=False),
)
def _gather_kernel(table_hbm, idx_hbm, out_hbm, idx_all, idx_t, rows, tbuf, sg, sw):
    wid = lax.axis_index("s") * NC + lax.axis_index("c")
    base = wid * B_PER_W
    pltpu.sync_copy(idx_hbm.at[pl.ds(base, B_PER_W)], idx_all)

    lane = lax.iota(jnp.int32, 16)
    lane50 = lane * HIST

    def idx_t_body(h, carry):
        for g in range(2 * NS):
            v = plsc.load_gather(idx_all, [lane50 + (g * 16 * HIST + h)])
            idx_t[h, g // 8, pl.ds((g % 8) * 16, 16)] = v
        return carry

    lax.fori_loop(0, HIST, idx_t_body, 0)

    def fire_gather(r, j):
        pltpu.async_copy(
            table_hbm.at[idx_t.at[r // BB_PER_W, r % BB_PER_W]], rows[j], sg[j]
        )

    def wait_gather(j):
        pltpu.make_async_copy(
            table_hbm.at[pl.ds(0, 128)], rows[j], sg[j]
        ).wait()

    def transpose_rows(j):
        def tbody(bg, carry):
            b0 = bg * 8
            for bi in range(8):
                b = b0 + bi
                bvec = jnp.full((16,), b, jnp.int32)
                for g in range(D // 16):
                    v = rows[j][b, pl.ds(g * 16, 16)]
                    plsc.store_scatter(tbuf[j], [lane + g * 16, bvec], v)
            return carry

        lax.fori_loop(0, 16, tbody, 0)

    def fire_write(r, j):
        h = r // BB_PER_W
        bbg = wid * BB_PER_W + r % BB_PER_W
        for db in range(8):
            pltpu.async_copy(
                tbuf[j].at[pl.ds(db * 8, 8)], out_hbm.at[h, db, bbg], sw[j]
            )

    def wait_write(j):
        for db in range(8):
            pltpu.make_async_copy(
                tbuf[j].at[pl.ds(0, 8)], out_hbm.at[0, 0, 0], sw[j]
            ).wait()

    fire_gather(0, 0)
    wait_gather(0)
    fire_gather(1, 1)
    transpose_rows(0)
    fire_write(0, 0)
    wait_gather(1)
    fire_gather(2, 0)
    transpose_rows(1)
    fire_write(1, 1)

    def body(i, carry):
        r = 2 * i
        wait_gather(0)
        fire_gather(r + 1, 1)
        wait_write(0)
        transpose_rows(0)
        fire_write(r, 0)
        wait_gather(1)
        fire_gather(r + 2, 0)
        wait_write(1)
        transpose_rows(1)
        fire_write(r + 1, 1)
        return carry

    lax.fori_loop(1, N_ROUNDS // 2 - 1, body, 0)

    rl = N_ROUNDS - 2
    wait_gather(0)
    fire_gather(rl + 1, 1)
    wait_write(0)
    transpose_rows(0)
    fire_write(rl, 0)
    wait_gather(1)
    wait_write(1)
    transpose_rows(1)
    fire_write(rl + 1, 1)

    wait_write(0)
    wait_write(1)


def kernel(x, table):
    idx = x.reshape(-1).astype(jnp.int32)
    o5 = _gather_kernel(table, idx)
    return o5.transpose(2, 4, 0, 1, 3).reshape(BATCH, HIST, D)


# speedup vs baseline: 1.2836x; 1.2836x over previous
import functools

import jax
import jax.numpy as jnp
from jax import lax
from jax.experimental import pallas as pl
from jax.experimental.pallas import tpu as pltpu
from jax.experimental.pallas import tpu_sc as plsc

VOCAB = 1000000
D = 64
B = 16384 * 50

_info = plsc.get_sparse_core_info()
NC, NS = _info.num_cores, _info.num_subcores
NW = NC * NS
B_PER_W = B // NW
CHUNK = 320
N_CHUNKS = B_PER_W // CHUNK
NBUF = 4
N_BLOCKS = N_CHUNKS // NBUF


@functools.partial(
    pl.kernel,
    mesh=plsc.VectorSubcoreMesh(core_axis_name="c", subcore_axis_name="s"),
    out_type=jax.ShapeDtypeStruct((B, D), jnp.float32),
    scratch_types=[
        pltpu.VMEM((B_PER_W,), jnp.int32),
        [pltpu.VMEM((CHUNK, D), jnp.float32) for _ in range(NBUF)],
        [pltpu.SemaphoreType.DMA for _ in range(NBUF)],
        [pltpu.SemaphoreType.DMA for _ in range(NBUF)],
    ],
    compiler_params=pltpu.CompilerParams(use_tc_tiling_on_sc=False),
)
def _gather_kernel(table_hbm, idx_hbm, out_hbm, idx_all, rows, sg, so):
    wid = lax.axis_index("s") * NC + lax.axis_index("c")
    base = wid * B_PER_W
    pltpu.sync_copy(idx_hbm.at[pl.ds(base, B_PER_W)], idx_all)

    def fire_gather(c, b):
        pltpu.async_copy(
            table_hbm.at[idx_all.at[pl.ds(c * CHUNK, CHUNK)]], rows[b], sg[b]
        )

    def wait_gather(b):
        pltpu.make_async_copy(
            out_hbm.at[pl.ds(base, CHUNK)], rows[b], sg[b]
        ).wait()

    def fire_write(c, b):
        pltpu.async_copy(rows[b], out_hbm.at[pl.ds(base + c * CHUNK, CHUNK)], so[b])

    def wait_write(b):
        pltpu.make_async_copy(
            rows[b], out_hbm.at[pl.ds(base, CHUNK)], so[b]
        ).wait()

    fire_gather(0, 0)
    fire_gather(1, 1)

    wait_gather(0)
    fire_write(0, 0)
    fire_gather(2, 2)
    wait_gather(1)
    fire_write(1, 1)
    fire_gather(3, 3)
    wait_gather(2)
    fire_write(2, 2)
    wait_write(0)
    fire_gather(4, 0)
    wait_gather(3)
    fire_write(3, 3)
    wait_write(1)
    fire_gather(5, 1)

    def body(i, carry):
        c0 = i * NBUF
        for b in range(NBUF):
            wait_gather(b)
            fire_write(c0 + b, b)
            wait_write((b + 2) % NBUF)
            fire_gather(c0 + b + 2, (b + 2) % NBUF)
        return carry

    lax.fori_loop(1, N_BLOCKS - 1, body, 0)

    cl = (N_BLOCKS - 1) * NBUF
    wait_gather(0)
    fire_write(cl, 0)
    wait_write(2)
    fire_gather(cl + 2, 2)
    wait_gather(1)
    fire_write(cl + 1, 1)
    wait_write(3)
    fire_gather(cl + 3, 3)
    wait_gather(2)
    fire_write(cl + 2, 2)
    wait_gather(3)
    fire_write(cl + 3, 3)

    for b in range(NBUF):
        wait_write(b)


def kernel(x, table):
    idx = x.reshape(-1).astype(jnp.int32)
    out = _gather_kernel(table, idx)
    return out.reshape(x.shape[0], x.shape[1], D)


# speedup vs baseline: 1.6365x; 1.2749x over previous
import functools

import jax
import jax.numpy as jnp
from jax import lax
from jax.experimental import pallas as pl
from jax.experimental.pallas import tpu as pltpu
from jax.experimental.pallas import tpu_sc as plsc

VOCAB = 1000000
D = 64
B = 16384 * 50

_info = plsc.get_sparse_core_info()
NC, NS = _info.num_cores, _info.num_subcores
NW = NC * NS
B_PER_W = B // NW
CHUNK = 320
N_CHUNKS = B_PER_W // CHUNK
NBUF = 4
N_BLOCKS = N_CHUNKS // NBUF


@functools.partial(
    pl.kernel,
    mesh=plsc.VectorSubcoreMesh(core_axis_name="c", subcore_axis_name="s"),
    out_type=jax.ShapeDtypeStruct((B, D), jnp.float32),
    scratch_types=[
        pltpu.VMEM((B_PER_W,), jnp.int32),
        [pltpu.VMEM((CHUNK, D), jnp.float32) for _ in range(NBUF)],
        [pltpu.SemaphoreType.DMA for _ in range(NBUF)],
        [pltpu.SemaphoreType.DMA for _ in range(NBUF)],
    ],
    compiler_params=pltpu.CompilerParams(use_tc_tiling_on_sc=False),
)
def _gather_kernel(table_hbm, idx_hbm, out_hbm, idx_all, rows, sg, so):
    wid = lax.axis_index("s") * NC + lax.axis_index("c")
    base = wid * B_PER_W
    pltpu.sync_copy(idx_hbm.at[pl.ds(base, B_PER_W)], idx_all)

    def fire_gather(c, b):
        pltpu.async_copy(
            table_hbm.at[idx_all.at[pl.ds(c * CHUNK, CHUNK)]], rows[b], sg[b]
        )

    def wait_gather(b):
        pltpu.make_async_copy(
            out_hbm.at[pl.ds(base, CHUNK)], rows[b], sg[b]
        ).wait()

    def fire_write(c, b):
        pltpu.async_copy(rows[b], out_hbm.at[pl.ds(base + c * CHUNK, CHUNK)], so[b])

    def wait_write(b):
        pltpu.make_async_copy(
            rows[b], out_hbm.at[pl.ds(base, CHUNK)], so[b]
        ).wait()

    fire_gather(0, 0)
    fire_gather(1, 1)

    wait_gather(0)
    fire_write(0, 0)
    fire_gather(2, 2)
    wait_gather(1)
    fire_write(1, 1)
    fire_gather(3, 3)
    wait_gather(2)
    fire_write(2, 2)
    wait_write(0)
    fire_gather(4, 0)
    wait_gather(3)
    fire_write(3, 3)
    wait_write(1)
    fire_gather(5, 1)

    def body(i, carry):
        c0 = i * NBUF
        for b in range(NBUF):
            wait_gather(b)
            fire_write(c0 + b, b)
            wait_write((b + 2) % NBUF)
            fire_gather(c0 + b + 2, (b + 2) % NBUF)
        return carry

    lax.fori_loop(1, N_BLOCKS - 1, body, 0)

    cl = (N_BLOCKS - 1) * NBUF
    wait_gather(0)
    fire_write(cl, 0)
    wait_write(2)
    fire_gather(cl + 2, 2)
    wait_gather(1)
    fire_write(cl + 1, 1)
    wait_write(3)
    fire_gather(cl + 3, 3)
    wait_gather(2)
    fire_write(cl + 2, 2)
    wait_gather(3)
    fire_write(cl + 3, 3)

    for b in range(NBUF):
        wait_write(b)


BATCH = 16384
HIST = 50
TBLK = 128 * HIST * D // 128


@functools.partial(
    pl.pallas_call,
    grid=(BATCH // 128,),
    in_specs=[pl.BlockSpec((TBLK, 128), lambda i: (i, 0))],
    out_specs=pl.BlockSpec((HIST, 8, 1, 8, 128), lambda i: (0, 0, i, 0, 0)),
    out_shape=jax.ShapeDtypeStruct((HIST, 8, BATCH // 128, 8, 128), jnp.float32),
)
def _retile_kernel(in_ref, out_ref):
    x3 = in_ref[...].reshape(128, HIST // 2, 128)
    for hh in range(HIST // 2):
        yt = x3[:, hh, :].T
        out_ref[2 * hh, :, 0, :, :] = yt[0:64].reshape(8, 8, 128)
        out_ref[2 * hh + 1, :, 0, :, :] = yt[64:128].reshape(8, 8, 128)


def kernel(x, table):
    idx = x.reshape(-1).astype(jnp.int32)
    out = _gather_kernel(table, idx)
    o5 = _retile_kernel(out.reshape(B * D // 128, 128))
    return o5.transpose(2, 4, 0, 1, 3).reshape(BATCH, HIST, D)
